# baseline (speedup 1.0000x reference)
import functools

import jax
import jax.numpy as jnp
import numpy as np
from jax import lax
from jax.experimental import pallas as pl
from jax.experimental.pallas import tpu as pltpu
from jax.experimental.pallas import tpu_sc as plsc

_MODALITY = (197, 50, 50, 50, 200, 105, 277, 61, 34)
_B, _S, _H = 16, 1024, 1024
_NTYPE = 9
_NTYPE_PAD = 16
_EPS = 1e-12
_GROUP = 2
_ROWS = _GROUP * _S
_NBLK = _B // _GROUP

_POS_IDS = np.concatenate([np.arange(L, dtype=np.int32) for L in _MODALITY])

_SC_CORES, _SC_SUBCORES = 2, 16
_NW = _SC_CORES * _SC_SUBCORES
_RPW = _S // _NW


def _sc_pos_gather(pos_table, idx):
    mesh = plsc.VectorSubcoreMesh(core_axis_name="c", subcore_axis_name="s")

    @functools.partial(
        pl.kernel, mesh=mesh,
        out_type=jax.ShapeDtypeStruct((_S, _H), jnp.float32),
        scratch_types=[
            pltpu.VMEM((_RPW,), jnp.int32),
            pltpu.VMEM((_RPW, _H), jnp.float32),
            pltpu.SemaphoreType.DMA,
        ],
    )
    def k(table_hbm, idx_hbm, out_hbm, idx_v, rows_v, sem):
        wid = lax.axis_index("s") * _SC_CORES + lax.axis_index("c")
        base = wid * _RPW
        pltpu.sync_copy(idx_hbm.at[pl.ds(base, _RPW)], idx_v)
        pltpu.async_copy(table_hbm.at[idx_v], rows_v, sem).wait()
        pltpu.sync_copy(rows_v, out_hbm.at[pl.ds(base, _RPW)])

    return k(pos_table, idx)


def _tc_kernel(tt_ref, x_ref, type_ref, pos_ref, gamma_ref, beta_ref, o_ref):
    ids = tt_ref[0]
    iota = jax.lax.broadcasted_iota(jnp.int32, (_NTYPE_PAD, _ROWS), 0)
    onehot = (iota == ids).astype(jnp.float32)
    type_emb = jax.lax.dot_general(
        onehot, type_ref[...],
        dimension_numbers=(((0,), (0,)), ((), ())),
        preferred_element_type=jnp.float32)

    gamma = gamma_ref[...]
    beta = beta_ref[...]
    for g in range(_GROUP):
        r0 = g * _S
        s = x_ref[0, r0:r0 + _S, :] + type_emb[r0:r0 + _S, :] + pos_ref[...]
        mean = jnp.mean(s, axis=1, keepdims=True)
        var = jnp.mean(s * s, axis=1, keepdims=True) - mean * mean
        inv = jax.lax.rsqrt(var + _EPS)
        o_ref[0, r0:r0 + _S, :] = (s - mean) * inv * gamma + beta


def kernel(inputs_embeds, token_type_ids, pos_table, type_table, ln_gamma, ln_beta):
    pos_emb = _sc_pos_gather(pos_table, jnp.asarray(_POS_IDS))

    x = inputs_embeds.reshape(_NBLK, _ROWS, _H)
    tt = token_type_ids.astype(jnp.int32).reshape(_NBLK, 1, _ROWS)
    type_pad = jnp.zeros((_NTYPE_PAD, _H), jnp.float32).at[:_NTYPE].set(
        type_table.astype(jnp.float32))
    gamma = ln_gamma.reshape(1, _H)
    beta = ln_beta.reshape(1, _H)

    out = pl.pallas_call(
        _tc_kernel,
        grid=(_NBLK,),
        in_specs=[
            pl.BlockSpec((1, 1, _ROWS), lambda b: (b, 0, 0)),
            pl.BlockSpec((1, _ROWS, _H), lambda b: (b, 0, 0)),
            pl.BlockSpec((_NTYPE_PAD, _H), lambda b: (0, 0)),
            pl.BlockSpec((_S, _H), lambda b: (0, 0)),
            pl.BlockSpec((1, _H), lambda b: (0, 0)),
            pl.BlockSpec((1, _H), lambda b: (0, 0)),
        ],
        out_specs=pl.BlockSpec((1, _ROWS, _H), lambda b: (b, 0, 0)),
        out_shape=jax.ShapeDtypeStruct((_NBLK, _ROWS, _H), jnp.float32),
    )(tt, x, type_pad, pos_emb, gamma, beta)
    return out.reshape(_B, _S, _H)
